# baseline (speedup 1.0000x reference)
import jax
import jax.numpy as jnp
from jax import lax
from jax.experimental import pallas as pl
from jax.experimental.pallas import tpu as pltpu

B = 8
Q = 900
T = 512
C = 400
K = 50
KPAD = 64
BIG = 2 ** 30
NEG = -3e38


def _body(logits_ref, boxes_ref, ts_ref, pmt_ref, mmat_ref, smat_ref,
          scores_ref, labels_ref, boxes_out_ref,
          prob_ref, rowmax_ref, cand_ref):
    pm = pmt_ref[...]
    for b in range(B):
        x = jax.nn.sigmoid(logits_ref[b, :, :])
        p = lax.dot_general(x, pm, (((1,), (1,)), ((), ())),
                            preferred_element_type=jnp.float32)
        prob_ref[b, :, :] = p
        rowmax_ref[b, :] = jnp.max(p, axis=1)

    ci = lax.broadcasted_iota(jnp.int32, (B, Q), 1)
    li = lax.broadcasted_iota(jnp.int32, (B, KPAD), 1)

    def p1(j, carry):
        rm, sel, v1 = carry
        m = jnp.max(rm, axis=1, keepdims=True)
        qidx = jnp.min(jnp.where(rm == m, ci, BIG), axis=1, keepdims=True)
        sel = jnp.where(li == j, qidx, sel)
        v1 = jnp.where(li == j, m, v1)
        rm = jnp.where(ci == qidx, NEG, rm)
        return rm, sel, v1

    sel0 = jnp.full((B, KPAD), -1, jnp.int32)
    v10 = jnp.full((B, KPAD), -1.0, jnp.float32)
    _, sel, rowbest0 = lax.fori_loop(0, K, p1, (rowmax_ref[...], sel0, v10),
                                     unroll=5)

    iq = lax.broadcasted_iota(jnp.int32, (B, KPAD, Q), 2)
    oh = (sel[:, :, None] == iq).astype(jnp.float32)
    for b in range(B):
        cand_ref[b, :, :] = jnp.dot(oh[b], prob_ref[b, :, :],
                                    preferred_element_type=jnp.float32,
                                    precision=jax.lax.Precision.HIGHEST)

    cc = lax.broadcasted_iota(jnp.int32, (B, C), 1)
    cc3 = lax.broadcasted_iota(jnp.int32, (B, KPAD, C), 2)

    def p2(i, carry):
        rowbest, vals, fidx = carry
        m = jnp.max(rowbest, axis=1, keepdims=True)
        r = jnp.min(jnp.where(rowbest == m, sel, BIG), axis=1, keepdims=True)
        ws = sel == r
        c = cand_ref[...]
        penal = jnp.where(ws, 0.0, NEG)
        wrow = jnp.max(c + penal[:, :, None], axis=1)
        col = jnp.min(jnp.where(wrow == m, cc, BIG), axis=1, keepdims=True)
        vals = jnp.where(li == i, m, vals)
        fidx = jnp.where(li == i, r * C + col, fidx)
        selmr = sel[:, :, None] - r[:, :, None]
        hit = (selmr | (cc3 - col[:, :, None])) == 0
        cand_ref[...] = jnp.where(hit, NEG, c)
        nb = jnp.max(jnp.where(cc == col, NEG, wrow), axis=1, keepdims=True)
        rowbest = jnp.where(ws, nb, rowbest)
        return rowbest, vals, fidx

    vals0 = jnp.zeros((B, KPAD), jnp.float32)
    fidx0 = jnp.zeros((B, KPAD), jnp.int32)
    _, vals, fidx = lax.fori_loop(0, K, p2, (rowbest0, vals0, fidx0),
                                  unroll=5)

    scores_ref[...] = vals[:, :K]
    labels_ref[...] = (fidx % C)[:, :K]

    qsel = fidx // C
    oh2 = (qsel[:, :, None] == iq).astype(jnp.float32)
    mmat = mmat_ref[...]
    smat = smat_ref[...]
    for b in range(B):
        g = jnp.dot(oh2[b], boxes_ref[b, :, :],
                    preferred_element_type=jnp.float32,
                    precision=jax.lax.Precision.HIGHEST)
        xy = jnp.dot(g, mmat, preferred_element_type=jnp.float32,
                     precision=jax.lax.Precision.HIGHEST)
        sc = jnp.dot(ts_ref[b:b + 1, :], smat,
                     preferred_element_type=jnp.float32,
                     precision=jax.lax.Precision.HIGHEST)
        boxes_out_ref[b, :, :] = (xy * sc)[:K, :]


def kernel(pred_logits, pred_boxes, target_sizes, positive_map):
    mmat = jnp.array([[1., 0., 1., 0.],
                      [0., 1., 0., 1.],
                      [-.5, 0., .5, 0.],
                      [0., -.5, 0., .5]], jnp.float32)
    smat = jnp.array([[0., 1., 0., 1.],
                      [1., 0., 1., 0.]], jnp.float32)
    scores, labels, boxes = pl.pallas_call(
        _body,
        out_shape=(
            jax.ShapeDtypeStruct((B, K), jnp.float32),
            jax.ShapeDtypeStruct((B, K), jnp.int32),
            jax.ShapeDtypeStruct((B, K, 4), jnp.float32),
        ),
        scratch_shapes=[
            pltpu.VMEM((B, Q, C), jnp.float32),
            pltpu.VMEM((B, Q), jnp.float32),
            pltpu.VMEM((B, KPAD, C), jnp.float32),
        ],
    )(pred_logits, pred_boxes, target_sizes, positive_map, mmat, smat)
    return (scores, labels, boxes)


# speedup vs baseline: 1.1710x; 1.1710x over previous
import jax
import jax.numpy as jnp
from jax import lax
from jax.experimental import pallas as pl
from jax.experimental.pallas import tpu as pltpu

B = 8
Q = 900
T = 512
C = 400
K = 50
KPAD = 64
BIG = 2 ** 30
NEG = -3e38


def _body(logits_ref, boxes_ref, ts_ref, pmt_ref, mmat_ref, smat_ref,
          scores_ref, labels_ref, boxes_out_ref,
          prob_ref, rowmax_ref, cand_ref, flat_ref):
    pm = pmt_ref[...]
    for b in range(B):
        x = jax.nn.sigmoid(logits_ref[b, :, :])
        p = lax.dot_general(x, pm, (((1,), (1,)), ((), ())),
                            preferred_element_type=jnp.float32)
        prob_ref[b, :, :] = p
        rowmax_ref[b, :] = jnp.max(p, axis=1)

    ci = lax.broadcasted_iota(jnp.int32, (B, Q), 1)
    li = lax.broadcasted_iota(jnp.int32, (B, KPAD), 1)

    def p1(j, carry):
        rm, sel = carry
        m = jnp.max(rm, axis=1, keepdims=True)
        qidx = jnp.min(jnp.where(rm == m, ci, BIG), axis=1, keepdims=True)
        sel = jnp.where(li == j, qidx, sel)
        rm = jnp.where(ci == qidx, NEG, rm)
        return rm, sel

    sel0 = jnp.full((B, KPAD), -1, jnp.int32)
    _, sel = lax.fori_loop(0, K, p1, (rowmax_ref[...], sel0), unroll=5)

    iq = lax.broadcasted_iota(jnp.int32, (B, KPAD, Q), 2)
    oh = (sel[:, :, None] == iq).astype(jnp.float32)
    rmask = lax.broadcasted_iota(jnp.int32, (KPAD, C), 0) < K
    for b in range(B):
        cb = jnp.dot(oh[b], prob_ref[b, :, :],
                     preferred_element_type=jnp.float32,
                     precision=jax.lax.Precision.HIGHEST)
        cand_ref[b, :, :] = jnp.where(rmask, cb, -1.0)
    flat_ref[...] = (sel[:, :, None] * C
                     + lax.broadcasted_iota(jnp.int32, (B, KPAD, C), 2))

    def p2(i, carry):
        vals, fidx = carry
        c = cand_ref[...]
        m = jnp.max(jnp.max(c, axis=2, keepdims=True), axis=1, keepdims=True)
        fl = flat_ref[...]
        fi = jnp.min(jnp.min(jnp.where(c == m, fl, BIG),
                             axis=2, keepdims=True), axis=1, keepdims=True)
        vals = jnp.where(li == i, m[:, :, 0], vals)
        fidx = jnp.where(li == i, fi[:, :, 0], fidx)
        cand_ref[...] = jnp.where(fl == fi, -2.0, c)
        return vals, fidx

    vals0 = jnp.zeros((B, KPAD), jnp.float32)
    fidx0 = jnp.zeros((B, KPAD), jnp.int32)
    vals, fidx = lax.fori_loop(0, K, p2, (vals0, fidx0), unroll=5)

    scores_ref[...] = vals[:, :K]
    labels_ref[...] = (fidx % C)[:, :K]

    qsel = fidx // C
    oh2 = (qsel[:, :, None] == iq).astype(jnp.float32)
    mmat = mmat_ref[...]
    smat = smat_ref[...]
    for b in range(B):
        g = jnp.dot(oh2[b], boxes_ref[b, :, :],
                    preferred_element_type=jnp.float32,
                    precision=jax.lax.Precision.HIGHEST)
        xy = jnp.dot(g, mmat, preferred_element_type=jnp.float32,
                     precision=jax.lax.Precision.HIGHEST)
        sc = jnp.dot(ts_ref[b:b + 1, :], smat,
                     preferred_element_type=jnp.float32,
                     precision=jax.lax.Precision.HIGHEST)
        boxes_out_ref[b, :, :] = (xy * sc)[:K, :]


def kernel(pred_logits, pred_boxes, target_sizes, positive_map):
    mmat = jnp.array([[1., 0., 1., 0.],
                      [0., 1., 0., 1.],
                      [-.5, 0., .5, 0.],
                      [0., -.5, 0., .5]], jnp.float32)
    smat = jnp.array([[0., 1., 0., 1.],
                      [1., 0., 1., 0.]], jnp.float32)
    scores, labels, boxes = pl.pallas_call(
        _body,
        out_shape=(
            jax.ShapeDtypeStruct((B, K), jnp.float32),
            jax.ShapeDtypeStruct((B, K), jnp.int32),
            jax.ShapeDtypeStruct((B, K, 4), jnp.float32),
        ),
        scratch_shapes=[
            pltpu.VMEM((B, Q, C), jnp.float32),
            pltpu.VMEM((B, Q), jnp.float32),
            pltpu.VMEM((B, KPAD, C), jnp.float32),
            pltpu.VMEM((B, KPAD, C), jnp.int32),
        ],
    )(pred_logits, pred_boxes, target_sizes, positive_map, mmat, smat)
    return (scores, labels, boxes)
